# baseline (speedup 1.0000x reference)
import jax
import jax.numpy as jnp
from jax.experimental import pallas as pl
from jax.experimental.pallas import tpu as pltpu

_B, _S, _D, _H = 2, 2048, 4096, 32
_Dh = _D // _H
_HALF = _Dh // 2
_BASE = 10000.0
_SCALE = _Dh ** -0.5
_BS = _B * _S
_NEG = -1e30

_BM_A = 1024
_BN_A = 256
_HEADS_PER_BLK = _BN_A // _Dh
_JPP = _D // _BN_A


def _qkv_rope_kernel(x_ref, w_ref, b_ref, cos_ref, sin_ref, o_ref):
    acc = jnp.dot(x_ref[...], w_ref[...], preferred_element_type=jnp.float32)
    acc = acc + b_ref[...]
    cos = cos_ref[0]
    sin = sin_ref[0]
    for a in range(_HEADS_PER_BLK):
        blk = acc[:, a * _Dh:(a + 1) * _Dh]
        rot = jnp.concatenate([blk[:, _HALF:], blk[:, :_HALF]], axis=1)
        o_ref[0, 0, a] = (blk * cos + rot * sin).astype(o_ref.dtype)


def _qkv_rope(x, wqkv, bias, cos_t, sin_t):
    grid = (_BS // _BM_A, 3 * _D // _BN_A)
    return pl.pallas_call(
        _qkv_rope_kernel,
        grid=grid,
        in_specs=[
            pl.BlockSpec((_BM_A, _D), lambda i, j: (i, 0)),
            pl.BlockSpec((_D, _BN_A), lambda i, j: (0, j)),
            pl.BlockSpec((1, _BN_A), lambda i, j: (0, j)),
            pl.BlockSpec((1, _BM_A, _Dh), lambda i, j: (j // _JPP, i, 0)),
            pl.BlockSpec((1, _BM_A, _Dh), lambda i, j: (j // _JPP, i, 0)),
        ],
        out_specs=pl.BlockSpec(
            (1, 1, _HEADS_PER_BLK, _BM_A, _Dh),
            lambda i, j: (j // _JPP, i // 2, j % _JPP, i % 2, 0),
        ),
        out_shape=jax.ShapeDtypeStruct((3, _B, _H, _S, _Dh), jnp.bfloat16),
        compiler_params=pltpu.CompilerParams(
            dimension_semantics=("parallel", "arbitrary"),
            vmem_limit_bytes=60000 * 1024,
        ),
        name="qkv_rope",
    )(x, wqkv, bias, cos_t, sin_t)


_BQ = 512
_NQ = _S // _BQ
_QH = 256


def _make_attn_kernel(qi):
    ext = (qi + 1) * _BQ

    def body(q_ref, k_ref, v_ref, mask_ref, o_ref):
        q = q_ref[0, 0, 0]
        k = k_ref[0, 0, 0]
        v = v_ref[0, 0, 0]
        s = jax.lax.dot_general(
            k, q, (((1,), (1,)), ((), ())),
            preferred_element_type=jnp.float32)
        if qi == 0:
            s = s + mask_ref[...]
        else:
            s = jnp.concatenate(
                [s[:qi * _BQ], s[qi * _BQ:] + mask_ref[...]], axis=0)
        m = jnp.max(s, axis=0, keepdims=True)
        p = jnp.exp2(s - m)
        l = jnp.sum(p, axis=0, keepdims=True)
        ctx_t = jax.lax.dot_general(
            v, p.astype(jnp.bfloat16), (((0,), (0,)), ((), ())),
            preferred_element_type=jnp.float32)
        o_ref[0] = (ctx_t * (1.0 / l)).astype(o_ref.dtype)

    return body


def _attention(qkvh, mask_diag):
    parts = []
    for qi in range(_NQ):
        ext = (qi + 1) * _BQ
        parts.append(pl.pallas_call(
            _make_attn_kernel(qi),
            grid=(_B * _H,),
            in_specs=[
                pl.BlockSpec((1, 1, 1, _BQ, _Dh),
                             lambda bh, qi=qi: (0, bh // _H, bh % _H, qi, 0)),
                pl.BlockSpec((1, 1, 1, ext, _Dh),
                             lambda bh: (1, bh // _H, bh % _H, 0, 0)),
                pl.BlockSpec((1, 1, 1, ext, _Dh),
                             lambda bh: (2, bh // _H, bh % _H, 0, 0)),
                pl.BlockSpec((_BQ, _BQ), lambda bh: (0, 0)),
            ],
            out_specs=pl.BlockSpec(
                (1, _Dh, _BQ), lambda bh: (bh // _H, bh % _H, 0)),
            out_shape=jax.ShapeDtypeStruct((_B, _D, _BQ), jnp.bfloat16),
            compiler_params=pltpu.CompilerParams(
                dimension_semantics=("parallel",),
                vmem_limit_bytes=60000 * 1024,
            ),
            name=f"attn_q{qi}",
        )(qkvh, qkvh, qkvh, mask_diag))
    return jnp.concatenate(parts, axis=2)


_BN_C = 256


def _proj_kernel(x_ref, w_ref, o_ref):
    w = w_ref[...].astype(jnp.bfloat16)
    o_ref[0] = jax.lax.dot_general(
        x_ref[0], w, (((0,), (0,)), ((), ())),
        preferred_element_type=jnp.float32)


def _out_proj(ctx_t, wproj):
    grid = (_B, _D // _BN_C)
    return pl.pallas_call(
        _proj_kernel,
        grid=grid,
        in_specs=[
            pl.BlockSpec((1, _D, _S), lambda i, j: (i, 0, 0)),
            pl.BlockSpec((_D, _BN_C), lambda i, j: (0, j)),
        ],
        out_specs=pl.BlockSpec((1, _S, _BN_C), lambda i, j: (i, 0, j)),
        out_shape=jax.ShapeDtypeStruct((_B, _S, _D), jnp.float32),
        compiler_params=pltpu.CompilerParams(
            dimension_semantics=("parallel", "arbitrary"),
            vmem_limit_bytes=60000 * 1024,
        ),
        name="out_proj",
    )(ctx_t, wproj)


def kernel(hidden_states, positions, Wqkv, bqkv, Wproj):
    x = hidden_states.reshape(_BS, _D)

    pos = positions.reshape(_BS).astype(jnp.float32)
    inv_freq = 1.0 / (_BASE ** (jnp.arange(_HALF, dtype=jnp.float32) / _HALF))
    ang = pos[:, None] * inv_freq[None, :]
    cos = jnp.cos(ang)
    sin = jnp.sin(ang)
    cos_f = jnp.concatenate([cos, cos], axis=1)
    sin_f = jnp.concatenate([-sin, sin], axis=1)
    ones = jnp.ones_like(cos_f)
    zeros = jnp.zeros_like(sin_f)
    qscale = _SCALE * 1.4426950408889634
    cos_t = jnp.stack([cos_f * qscale, cos_f, ones])
    sin_t = jnp.stack([sin_f * qscale, sin_f, zeros])

    k_idx = jnp.arange(_BQ, dtype=jnp.int32)[:, None]
    q_idx = jnp.arange(_BQ, dtype=jnp.int32)[None, :]
    mask_diag = jnp.where(q_idx >= k_idx, 0.0, _NEG).astype(jnp.float32)

    qkvh = _qkv_rope(x, Wqkv, bqkv.reshape(1, 3 * _D), cos_t, sin_t)
    ctx_t = _attention(qkvh, mask_diag)
    out = _out_proj(ctx_t, Wproj)
    return out.reshape(_B, _S, _D)
